# Initial kernel scaffold; baseline (speedup 1.0000x reference)
import jax
import jax.numpy as jnp
from jax.experimental import pallas as pl


def kernel(bpe_features, word_ids, layer_w):
    raise NotImplementedError("write your pallas kernel here")


# speedup vs baseline: 3.9769x; 3.9769x over previous
import jax
import jax.numpy as jnp
from jax import lax
from jax.experimental import pallas as pl
from jax.experimental.pallas import tpu as pltpu

B, L, S, D, W = 16, 13, 512, 768, 256
SB = 256


def _mix_segsum_kernel(w_ref, ids_ref, x_ref, out_ref):
    sb = pl.program_id(1)

    wv = w_ref[0, :]
    wv = wv - jnp.max(wv)
    ev = jnp.exp(wv)
    wn = ev / jnp.sum(ev)

    att = x_ref[0, 0] * wn[0]
    for l in range(1, L):
        att = att + x_ref[0, l] * wn[l]

    ids = ids_ref[0, 0, :]
    onehot = (ids[:, None] == lax.broadcasted_iota(jnp.int32, (SB, W), 1)
              ).astype(jnp.float32)
    contrib = lax.dot_general(
        onehot, att, (((0,), (0,)), ((), ())),
        preferred_element_type=jnp.float32,
        precision=lax.Precision.HIGHEST)

    @pl.when(sb == 0)
    def _():
        out_ref[0] = jnp.zeros_like(out_ref[0])

    out_ref[0] += contrib


def kernel(bpe_features, word_ids, layer_w):
    ids3 = word_ids.reshape(B, 1, S)
    w2 = layer_w.reshape(1, L)
    grid = (B, S // SB)
    return pl.pallas_call(
        _mix_segsum_kernel,
        grid=grid,
        in_specs=[
            pl.BlockSpec((1, L), lambda b, s: (0, 0)),
            pl.BlockSpec((1, 1, SB), lambda b, s: (b, 0, s)),
            pl.BlockSpec((1, L, SB, D), lambda b, s: (b, 0, s, 0)),
        ],
        out_specs=pl.BlockSpec((1, W, D), lambda b, s: (b, 0, 0)),
        out_shape=jax.ShapeDtypeStruct((B, W, D), jnp.float32),
        compiler_params=pltpu.CompilerParams(
            dimension_semantics=("parallel", "arbitrary")),
    )(w2, ids3, bpe_features)
